# Initial kernel scaffold; baseline (speedup 1.0000x reference)
import jax
import jax.numpy as jnp
from jax.experimental import pallas as pl


def kernel(x, pos_embedding):
    raise NotImplementedError("write your pallas kernel here")


# speedup vs baseline: 1.5939x; 1.5939x over previous
import jax
import jax.numpy as jnp
from jax.experimental import pallas as pl


_SEQ_BLOCK = 256


def _add_kernel(x_ref, pos_ref, out_ref):
    out_ref[...] = x_ref[...] + pos_ref[...][None, :, :]


def kernel(x, pos_embedding):
    batch, seq_len, hidden = x.shape
    pos = pos_embedding[:seq_len]
    blk = _SEQ_BLOCK
    if seq_len % blk != 0:
        blk = seq_len
    grid = (seq_len // blk,)
    return pl.pallas_call(
        _add_kernel,
        grid=grid,
        in_specs=[
            pl.BlockSpec((batch, blk, hidden), lambda i: (0, i, 0)),
            pl.BlockSpec((blk, hidden), lambda i: (i, 0)),
        ],
        out_specs=pl.BlockSpec((batch, blk, hidden), lambda i: (0, i, 0)),
        out_shape=jax.ShapeDtypeStruct((batch, seq_len, hidden), x.dtype),
    )(x, pos)
